# baseline (device time: 132863 ns/iter reference)
import functools

import jax
import jax.numpy as jnp
from jax import lax
from jax.experimental import pallas as pl
from jax.experimental.pallas import tpu as pltpu

N_DEV = 8
SQ = 512
D = 1024
DH = 128
H_LOC = 8
SCALE = 0.08838834764831843

NR_AG = 3
NL_AG = 3
NR_RS = 3
NL_RS = 3


def kernel(x, Wq, Wo, Wk, Wv):
    def body(x_ref, wq_ref, wo_ref, wk_ref, wv_ref, out_ref,
             xall, parts, rsr, rsl, dbuf,
             agr_ss, agl_ss, ag_rs, rsr_ss, rsr_rs, rsl_ss, rsl_rs,
             d_ss, d_rs, dx_ss):
        dm = lax.axis_index("i")

        def ring(p):
            return jnp.where(p >= 4, lax.bitwise_xor(p, 3), p)

        d = ring(dm)
        right = ring(lax.rem(d + 1, N_DEV))
        left = ring(lax.rem(d + N_DEV - 1, N_DEV))

        def off(k):
            return lax.rem(d + (k % N_DEV), N_DEV)

        anti = ring(off(4))

        barrier_sem = pltpu.get_barrier_semaphore()
        for nbr in (left, right, anti):
            pl.semaphore_signal(
                barrier_sem, inc=1,
                device_id=(nbr,), device_id_type=pl.DeviceIdType.MESH,
            )
        pl.semaphore_wait(barrier_sem, 3)

        wq_bf = wq_ref[...].astype(jnp.bfloat16)
        wk_bf = wk_ref[...].astype(jnp.bfloat16)
        wv_bf = wv_ref[...].astype(jnp.bfloat16)
        wo_bf = wo_ref[...].astype(jnp.bfloat16)

        def compute_part(k):
            xb = xall[off(k)]
            q = jnp.dot(xb, wq_bf, preferred_element_type=jnp.float32)
            kk = jnp.dot(xb, wk_bf, preferred_element_type=jnp.float32)
            v = jnp.dot(xb, wv_bf, preferred_element_type=jnp.float32)
            q = q.astype(jnp.bfloat16)
            kk = kk.astype(jnp.bfloat16)
            v = v.astype(jnp.bfloat16)
            outs = []
            for h in range(H_LOC):
                sl = slice(h * DH, (h + 1) * DH)
                qh, kh, vh = q[:, sl], kk[:, sl], v[:, sl]
                s = lax.dot_general(
                    qh, kh, (((1,), (1,)), ((), ())),
                    preferred_element_type=jnp.float32,
                ) * SCALE
                p = jnp.exp(s)
                rinv = 1.0 / jnp.sum(p, axis=1, keepdims=True)
                pv = jnp.dot(
                    p.astype(jnp.bfloat16), vh,
                    preferred_element_type=jnp.float32,
                )
                outs.append((pv * rinv).astype(jnp.bfloat16))
            attn = jnp.concatenate(outs, axis=1)
            part = jnp.dot(attn, wo_bf, preferred_element_type=jnp.float32)
            parts[off(k)] = part.astype(jnp.bfloat16)

        def ag_send(hop, slot, dev):
            sems = agr_ss if dev is right else agl_ss
            return pltpu.make_async_remote_copy(
                src_ref=xall.at[slot], dst_ref=xall.at[slot],
                send_sem=sems.at[hop], recv_sem=ag_rs.at[slot],
                device_id=(dev,), device_id_type=pl.DeviceIdType.MESH,
            )

        def ag_wait_recv(slot):
            pltpu.make_async_remote_copy(
                src_ref=xall.at[slot], dst_ref=xall.at[slot],
                send_sem=agr_ss.at[0], recv_sem=ag_rs.at[slot],
                device_id=(right,), device_id_type=pl.DeviceIdType.MESH,
            ).wait_recv()

        def rsr_copy(step):
            src = parts.at[off(3)] if step == 0 else rsr.at[step - 1]
            return pltpu.make_async_remote_copy(
                src_ref=src, dst_ref=rsr.at[step],
                send_sem=rsr_ss.at[step], recv_sem=rsr_rs.at[step],
                device_id=(right,), device_id_type=pl.DeviceIdType.MESH,
            )

        def direct_copy():
            return pltpu.make_async_remote_copy(
                src_ref=parts.at[off(4)], dst_ref=dbuf,
                send_sem=d_ss.at[0], recv_sem=d_rs.at[0],
                device_id=(ring(off(4)),),
                device_id_type=pl.DeviceIdType.MESH,
            )

        def rsl_copy(step):
            src = parts.at[off(-3)] if step == 0 else rsl.at[step - 1]
            return pltpu.make_async_remote_copy(
                src_ref=src, dst_ref=rsl.at[step],
                send_sem=rsl_ss.at[step], recv_sem=rsl_rs.at[step],
                device_id=(left,), device_id_type=pl.DeviceIdType.MESH,
            )

        def fold(buf, step, k):
            buf[step] = (
                buf[step].astype(jnp.float32)
                + parts[off(k)].astype(jnp.float32)
            ).astype(jnp.bfloat16)

        def xdirect_copy():
            return pltpu.make_async_remote_copy(
                src_ref=xall.at[d], dst_ref=xall.at[d],
                send_sem=dx_ss.at[0], recv_sem=ag_rs.at[d],
                device_id=(anti,), device_id_type=pl.DeviceIdType.MESH,
            )

        xall[d] = x_ref[0].astype(jnp.bfloat16)
        ag_send(0, d, right).start()
        ag_send(0, d, left).start()
        xdirect_copy().start()
        compute_part(0)

        ag_wait_recv(off(-1))
        ag_send(1, off(-1), right).start()
        ag_wait_recv(off(1))
        ag_send(1, off(1), left).start()
        compute_part(-1)
        ag_wait_recv(off(-2))
        ag_send(2, off(-2), right).start()
        ag_wait_recv(off(2))
        ag_send(2, off(2), left).start()
        compute_part(-2)
        ag_wait_recv(off(-3))
        ag_wait_recv(off(3))
        compute_part(3)
        rsr_copy(0).start()
        compute_part(-3)
        rsl_copy(0).start()
        compute_part(2)
        rsr_copy(0).wait_recv()
        fold(rsr, 0, 2)
        rsr_copy(1).start()
        compute_part(1)
        rsl_copy(0).wait_recv()
        fold(rsl, 0, -2)
        rsl_copy(1).start()
        ag_wait_recv(off(4))
        compute_part(4)
        direct_copy().start()
        rsr_copy(1).wait_recv()
        fold(rsr, 1, 1)
        rsr_copy(2).start()
        rsl_copy(1).wait_recv()
        fold(rsl, 1, -1)
        rsl_copy(2).start()
        rsr_copy(2).wait_recv()
        rsl_copy(2).wait_recv()
        direct_copy().wait_recv()
        out_ref[0] = (
            rsr[NR_RS - 1].astype(jnp.float32)
            + rsl[NL_RS - 1].astype(jnp.float32)
            + dbuf[...].astype(jnp.float32)
            + parts[d].astype(jnp.float32)
        )

        for h in range(NR_AG):
            ag_send(h, off(-h), right).wait_send()
        for h in range(NL_AG):
            ag_send(h, off(h), left).wait_send()
        for s in range(NR_RS):
            rsr_copy(s).wait_send()
        for s in range(NL_RS):
            rsl_copy(s).wait_send()
        direct_copy().wait_send()
        xdirect_copy().wait_send()

        @functools.partial(
            pl.run_scoped, second_barrier=pltpu.SemaphoreType.REGULAR
        )
        def _(second_barrier):
            for nbr in (left, right, anti):
                pl.semaphore_signal(
                    second_barrier, inc=1,
                    device_id=(nbr,), device_id_type=pl.DeviceIdType.MESH,
                )
            pl.semaphore_wait(second_barrier, 3)

    return pl.pallas_call(
        body,
        out_shape=jax.ShapeDtypeStruct((1, SQ, D), jnp.float32),
        in_specs=[pl.BlockSpec(memory_space=pltpu.VMEM)] * 5,
        out_specs=pl.BlockSpec(memory_space=pltpu.VMEM),
        scratch_shapes=[
            pltpu.VMEM((N_DEV, SQ, D), jnp.bfloat16),
            pltpu.VMEM((N_DEV, SQ, D), jnp.bfloat16),
            pltpu.VMEM((NR_RS, SQ, D), jnp.bfloat16),
            pltpu.VMEM((NL_RS, SQ, D), jnp.bfloat16),
            pltpu.VMEM((SQ, D), jnp.bfloat16),
            pltpu.SemaphoreType.DMA((NR_AG,)),
            pltpu.SemaphoreType.DMA((NL_AG,)),
            pltpu.SemaphoreType.DMA((N_DEV,)),
            pltpu.SemaphoreType.DMA((NR_RS,)),
            pltpu.SemaphoreType.DMA((NR_RS,)),
            pltpu.SemaphoreType.DMA((NL_RS,)),
            pltpu.SemaphoreType.DMA((NL_RS,)),
            pltpu.SemaphoreType.DMA((1,)),
            pltpu.SemaphoreType.DMA((1,)),
            pltpu.SemaphoreType.DMA((1,)),
        ],
        compiler_params=pltpu.CompilerParams(
            collective_id=0, vmem_limit_bytes=100 * 1024 * 1024
        ),
    )(x, Wq, Wo, Wk, Wv)


# device time: 118911 ns/iter; 1.1173x vs baseline; 1.1173x over previous
import functools

import jax
import jax.numpy as jnp
from jax import lax
from jax.experimental import pallas as pl
from jax.experimental.pallas import tpu as pltpu

N_DEV = 8
SQ = 512
D = 1024
DH = 128
H_LOC = 8
SCALE = 0.08838834764831843

NR_AG = 3
NL_AG = 4
NR_RS = 3
NL_RS = 3


def kernel(x, Wq, Wo, Wk, Wv):
    def body(x_ref, wq_ref, wo_ref, wk_ref, wv_ref, out_ref,
             xall, parts, rsr, rsl, dbuf,
             agr_ss, agl_ss, ag_rs, rsr_ss, rsr_rs, rsl_ss, rsl_rs,
             d_ss, d_rs):
        dm = lax.axis_index("i")

        def ring(p):
            return jnp.where(p >= 4, lax.bitwise_xor(p, 3), p)

        d = ring(dm)
        right = ring(lax.rem(d + 1, N_DEV))
        left = ring(lax.rem(d + N_DEV - 1, N_DEV))

        def off(k):
            return lax.rem(d + (k % N_DEV), N_DEV)

        barrier_sem = pltpu.get_barrier_semaphore()
        for nbr in (left, right):
            pl.semaphore_signal(
                barrier_sem, inc=1,
                device_id=(nbr,), device_id_type=pl.DeviceIdType.MESH,
            )
        pl.semaphore_wait(barrier_sem, 2)

        wq_bf = wq_ref[...].astype(jnp.bfloat16)
        wk_bf = wk_ref[...].astype(jnp.bfloat16)
        wv_bf = wv_ref[...].astype(jnp.bfloat16)
        wo_bf = wo_ref[...].astype(jnp.bfloat16)

        def compute_part(k):
            xb = xall[off(k)]
            q = jnp.dot(xb, wq_bf, preferred_element_type=jnp.float32)
            kk = jnp.dot(xb, wk_bf, preferred_element_type=jnp.float32)
            v = jnp.dot(xb, wv_bf, preferred_element_type=jnp.float32)
            q = q.astype(jnp.bfloat16)
            kk = kk.astype(jnp.bfloat16)
            v = v.astype(jnp.bfloat16)
            outs = []
            for h in range(H_LOC):
                sl = slice(h * DH, (h + 1) * DH)
                qh, kh, vh = q[:, sl], kk[:, sl], v[:, sl]
                s = lax.dot_general(
                    qh, kh, (((1,), (1,)), ((), ())),
                    preferred_element_type=jnp.float32,
                ) * SCALE
                p = jnp.exp(s)
                rinv = 1.0 / jnp.sum(p, axis=1, keepdims=True)
                pv = jnp.dot(
                    p.astype(jnp.bfloat16), vh,
                    preferred_element_type=jnp.float32,
                )
                outs.append((pv * rinv).astype(jnp.bfloat16))
            attn = jnp.concatenate(outs, axis=1)
            part = jnp.dot(attn, wo_bf, preferred_element_type=jnp.float32)
            parts[off(k)] = part.astype(jnp.bfloat16)

        def rows(q):
            return pl.ds(q * (SQ // 2), SQ // 2)

        def ag_send(hop, slot, dev, q):
            sems = agr_ss if dev is right else agl_ss
            return pltpu.make_async_remote_copy(
                src_ref=xall.at[slot, rows(q)],
                dst_ref=xall.at[slot, rows(q)],
                send_sem=sems.at[hop, q], recv_sem=ag_rs.at[slot, q],
                device_id=(dev,), device_id_type=pl.DeviceIdType.MESH,
            )

        def ag_wait_recv(slot, q):
            pltpu.make_async_remote_copy(
                src_ref=xall.at[slot, rows(q)],
                dst_ref=xall.at[slot, rows(q)],
                send_sem=agr_ss.at[0, q], recv_sem=ag_rs.at[slot, q],
                device_id=(right,), device_id_type=pl.DeviceIdType.MESH,
            ).wait_recv()

        def rsr_copy(step, q):
            src = (
                parts.at[off(3), rows(q)] if step == 0
                else rsr.at[step - 1, rows(q)]
            )
            return pltpu.make_async_remote_copy(
                src_ref=src, dst_ref=rsr.at[step, rows(q)],
                send_sem=rsr_ss.at[step, q], recv_sem=rsr_rs.at[step, q],
                device_id=(right,), device_id_type=pl.DeviceIdType.MESH,
            )

        def direct_copy():
            return pltpu.make_async_remote_copy(
                src_ref=parts.at[off(4)], dst_ref=dbuf,
                send_sem=d_ss.at[0], recv_sem=d_rs.at[0],
                device_id=(ring(off(4)),),
                device_id_type=pl.DeviceIdType.MESH,
            )

        def rsl_copy(step, q):
            src = (
                parts.at[off(-3), rows(q)] if step == 0
                else rsl.at[step - 1, rows(q)]
            )
            return pltpu.make_async_remote_copy(
                src_ref=src, dst_ref=rsl.at[step, rows(q)],
                send_sem=rsl_ss.at[step, q], recv_sem=rsl_rs.at[step, q],
                device_id=(left,), device_id_type=pl.DeviceIdType.MESH,
            )

        def fold(buf, step, k, q):
            buf[step, rows(q)] = (
                buf[step, rows(q)].astype(jnp.float32)
                + parts[off(k), rows(q)].astype(jnp.float32)
            ).astype(jnp.bfloat16)

        def rs_relay(copy_fn, step, k):
            for q in (0, 1):
                copy_fn(step, q).wait_recv()
                fold(rsr if copy_fn is rsr_copy else rsl, step, k, q)
                copy_fn(step + 1, q).start()

        xall[d] = x_ref[0].astype(jnp.bfloat16)
        for q in (0, 1):
            ag_send(0, d, right, q).start()
            ag_send(0, d, left, q).start()
        compute_part(0)

        for q in (0, 1):
            ag_wait_recv(off(-1), q)
            ag_send(1, off(-1), right, q).start()
        for q in (0, 1):
            ag_wait_recv(off(1), q)
            ag_send(1, off(1), left, q).start()
        compute_part(-1)
        for q in (0, 1):
            ag_wait_recv(off(-2), q)
            ag_send(2, off(-2), right, q).start()
        for q in (0, 1):
            ag_wait_recv(off(2), q)
            ag_send(2, off(2), left, q).start()
        compute_part(-2)
        for q in (0, 1):
            ag_wait_recv(off(-3), q)
        for q in (0, 1):
            ag_wait_recv(off(3), q)
            ag_send(3, off(3), left, q).start()
        compute_part(3)
        for q in (0, 1):
            rsr_copy(0, q).start()
        compute_part(-3)
        for q in (0, 1):
            rsl_copy(0, q).start()
        compute_part(2)
        rs_relay(rsr_copy, 0, 2)
        for q in (0, 1):
            ag_wait_recv(off(4), q)
        compute_part(4)
        direct_copy().start()
        rs_relay(rsl_copy, 0, -2)
        compute_part(1)
        rs_relay(rsr_copy, 1, 1)
        rs_relay(rsl_copy, 1, -1)
        for q in (0, 1):
            rsr_copy(2, q).wait_recv()
            rsl_copy(2, q).wait_recv()
        direct_copy().wait_recv()
        out_ref[0] = (
            rsr[NR_RS - 1].astype(jnp.float32)
            + rsl[NL_RS - 1].astype(jnp.float32)
            + dbuf[...].astype(jnp.float32)
            + parts[d].astype(jnp.float32)
        )

        for h in range(NR_AG):
            for q in (0, 1):
                ag_send(h, off(-h), right, q).wait_send()
        for h in range(NL_AG):
            for q in (0, 1):
                ag_send(h, off(h), left, q).wait_send()
        for s in range(NR_RS):
            for q in (0, 1):
                rsr_copy(s, q).wait_send()
        for s in range(NL_RS):
            for q in (0, 1):
                rsl_copy(s, q).wait_send()
        direct_copy().wait_send()

        @functools.partial(
            pl.run_scoped, second_barrier=pltpu.SemaphoreType.REGULAR
        )
        def _(second_barrier):
            for nbr in (left, right):
                pl.semaphore_signal(
                    second_barrier, inc=1,
                    device_id=(nbr,), device_id_type=pl.DeviceIdType.MESH,
                )
            pl.semaphore_wait(second_barrier, 2)

    return pl.pallas_call(
        body,
        out_shape=jax.ShapeDtypeStruct((1, SQ, D), jnp.float32),
        in_specs=[pl.BlockSpec(memory_space=pltpu.VMEM)] * 5,
        out_specs=pl.BlockSpec(memory_space=pltpu.VMEM),
        scratch_shapes=[
            pltpu.VMEM((N_DEV, SQ, D), jnp.bfloat16),
            pltpu.VMEM((N_DEV, SQ, D), jnp.bfloat16),
            pltpu.VMEM((NR_RS, SQ, D), jnp.bfloat16),
            pltpu.VMEM((NL_RS, SQ, D), jnp.bfloat16),
            pltpu.VMEM((SQ, D), jnp.bfloat16),
            pltpu.SemaphoreType.DMA((NR_AG, 2)),
            pltpu.SemaphoreType.DMA((NL_AG, 2)),
            pltpu.SemaphoreType.DMA((N_DEV, 2)),
            pltpu.SemaphoreType.DMA((NR_RS, 2)),
            pltpu.SemaphoreType.DMA((NR_RS, 2)),
            pltpu.SemaphoreType.DMA((NL_RS, 2)),
            pltpu.SemaphoreType.DMA((NL_RS, 2)),
            pltpu.SemaphoreType.DMA((1,)),
            pltpu.SemaphoreType.DMA((1,)),
        ],
        compiler_params=pltpu.CompilerParams(
            collective_id=0, vmem_limit_bytes=100 * 1024 * 1024
        ),
    )(x, Wq, Wo, Wk, Wv)
